# Initial kernel scaffold; baseline (speedup 1.0000x reference)
import jax
import jax.numpy as jnp
from jax.experimental import pallas as pl


def kernel(lig_x, lig_pos, lig_edge_index, rec_x, rec_pos, rec_edge_index, params):
    raise NotImplementedError("write your pallas kernel here")


# speedup vs baseline: 1.6283x; 1.6283x over previous
import functools
import math

import jax
from jax import numpy as jnp
from jax import lax
from jax.experimental import pallas as pl
from jax.experimental.pallas import tpu as pltpu
from jax.experimental.pallas import tpu_sc as plsc

D = 128
EH = 64
A = 64
KQ = 32
NLIG = 2000
NREC = 8000
ELIG = 64000
EREC = 256000

NC = 2
NS = 16
NW = NC * NS
CHK = 128

TW = 256
VW = 128

ELP = 65536
ERP = 262144
NL16 = 2048
NR16 = 8064


def _silu(x):
    return x * jax.nn.sigmoid(x)


def _b16(x):
    return x.astype(jnp.bfloat16).astype(jnp.float32)


_PREC = None
_HI = lax.Precision.HIGHEST


def _dot(a, b, prec=_PREC):
    return jnp.dot(a, b, preferred_element_type=jnp.float32, precision=prec)


def _dot_nt(a, b, prec=_PREC):
    return lax.dot_general(a, b, (((1,), (1,)), ((), ())),
                           preferred_element_type=jnp.float32, precision=prec)


def _make_gather(ep, nch):
    mesh = plsc.VectorSubcoreMesh(core_axis_name="c", subcore_axis_name="s")

    @functools.partial(
        pl.kernel,
        out_type=[jax.ShapeDtypeStruct((ep, TW), jnp.float32),
                  jax.ShapeDtypeStruct((ep, TW), jnp.float32)],
        mesh=mesh,
        scratch_types=[pltpu.VMEM((nch, CHK), jnp.int32),
                       pltpu.VMEM((nch, CHK), jnp.int32),
                       pltpu.VMEM((CHK, TW), jnp.float32),
                       pltpu.VMEM((CHK, TW), jnp.float32),
                       pltpu.SemaphoreType.DMA,
                       pltpu.SemaphoreType.DMA])
    def gather_kernel(tab_hbm, idxd_hbm, idxs_hbm, outd_hbm, outs_hbm,
                      idxd_v, idxs_v, rowd_v, rows_v, semd, sems):
        wid = lax.axis_index("s") * NC + lax.axis_index("c")
        pltpu.sync_copy(idxd_hbm.at[wid], idxd_v)
        pltpu.sync_copy(idxs_hbm.at[wid], idxs_v)
        base = wid * nch * CHK

        def body(j, carry):
            cpd = pltpu.async_copy(tab_hbm.at[idxd_v.at[j]], rowd_v, semd)
            cps = pltpu.async_copy(tab_hbm.at[idxs_v.at[j]], rows_v, sems)
            cpd.wait()
            cps.wait()
            pltpu.sync_copy(rowd_v, outd_hbm.at[pl.ds(base + j * CHK, CHK)])
            pltpu.sync_copy(rows_v, outs_hbm.at[pl.ds(base + j * CHK, CHK)])
            return carry

        lax.fori_loop(0, nch, body, 0)

    return gather_kernel


def _make_scatter(n16, nch):
    mesh = plsc.VectorSubcoreMesh(core_axis_name="c", subcore_axis_name="s")
    rpt = n16 // NS

    @functools.partial(
        pl.kernel,
        out_type=[jax.ShapeDtypeStruct((NC, n16, VW), jnp.float32)],
        mesh=mesh,
        scratch_types=[pltpu.VMEM((nch, CHK), jnp.int32),
                       pltpu.VMEM((CHK, VW), jnp.float32),
                       pltpu.VMEM_SHARED((n16, VW), jnp.float32),
                       pltpu.SemaphoreType.DMA])
    def scatter_kernel(vals_hbm, idx_hbm, zeros_hbm, out_hbm, idx_v, vals_v,
                       acc_sh, sem):
        c = lax.axis_index("c")
        s = lax.axis_index("s")
        wid = s * NC + c
        pltpu.sync_copy(zeros_hbm.at[pl.ds(s * rpt, rpt)],
                        acc_sh.at[pl.ds(s * rpt, rpt)])
        plsc.subcore_barrier()
        pltpu.sync_copy(idx_hbm.at[wid], idx_v)
        base = wid * nch * CHK

        def body(j, carry):
            pltpu.sync_copy(vals_hbm.at[pl.ds(base + j * CHK, CHK)], vals_v)
            pltpu.sync_copy(vals_v, acc_sh.at[idx_v.at[j]], add=True)
            return carry

        lax.fori_loop(0, nch, body, 0)
        plsc.subcore_barrier()
        pltpu.sync_copy(acc_sh.at[pl.ds(s * rpt, rpt)],
                        out_hbm.at[c, pl.ds(s * rpt, rpt)])

    return scatter_kernel


def _make_edge_mlp(ep, be, with_aux):
    def body(gd_ref, gs_ref, we1_ref, wdist_ref, be1_ref, we2_ref, be2_ref,
             wc1_ref, bc1_ref, wc2_ref, bc2_ref, *out_refs):
        gd = gd_ref[...]
        gs = gs_ref[...]
        nb = gd.shape[0]
        cat = jnp.concatenate([gd[:, :D], gs[:, :D]], axis=1)
        rel = gd[:, D:D + 3] - gs[:, D:D + 3]
        dist2 = jnp.sum(rel * rel, axis=1, keepdims=True)
        pre = (_dot(cat, we1_ref[...])
               + _b16(dist2) * _b16(wdist_ref[...]) + be1_ref[...])
        h = _silu(pre)
        m = _silu(_dot(h, we2_ref[...]) + be2_ref[...])
        out_refs[0][...] = m
        if with_aux:
            c1 = _silu(_dot(m, wc1_ref[...]) + bc1_ref[...])
            cw = (jnp.sum(_b16(c1) * _b16(wc2_ref[...]), axis=1, keepdims=True)
                  + bc2_ref[...])
            ones = jnp.ones((nb, 1), jnp.float32)
            pad = jnp.zeros((nb, VW - 4), jnp.float32)
            out_refs[1][...] = jnp.concatenate([rel * cw, ones, pad], axis=1)

    grid = ep // be
    espec = pl.BlockSpec((be, VW), lambda i: (i, 0))
    eshape = jax.ShapeDtypeStruct((ep, VW), jnp.float32)
    return pl.pallas_call(
        body,
        grid=(grid,),
        in_specs=[pl.BlockSpec((be, TW), lambda i: (i, 0)),
                  pl.BlockSpec((be, TW), lambda i: (i, 0)),
                  pl.BlockSpec((2 * D, EH), lambda i: (0, 0)),
                  pl.BlockSpec((1, EH), lambda i: (0, 0)),
                  pl.BlockSpec((1, EH), lambda i: (0, 0)),
                  pl.BlockSpec((EH, D), lambda i: (0, 0)),
                  pl.BlockSpec((1, D), lambda i: (0, 0)),
                  pl.BlockSpec((D, EH), lambda i: (0, 0)),
                  pl.BlockSpec((1, EH), lambda i: (0, 0)),
                  pl.BlockSpec((1, EH), lambda i: (0, 0)),
                  pl.BlockSpec((1, 1), lambda i: (0, 0))],
        out_specs=[espec, espec] if with_aux else espec,
        out_shape=[eshape, eshape] if with_aux else eshape)


def _make_qkv(n, bn):
    def body(x_ref, w_ref, out_ref):
        out_ref[...] = _dot(x_ref[...], w_ref[...])

    return pl.pallas_call(
        body,
        grid=(n // bn,),
        in_specs=[pl.BlockSpec((bn, D), lambda i: (i, 0)),
                  pl.BlockSpec((D, 3 * A), lambda i: (0, 0))],
        out_specs=pl.BlockSpec((bn, 3 * A), lambda i: (i, 0)),
        out_shape=jax.ShapeDtypeStruct((n, 3 * A), jnp.float32))


def _make_attn(nq, bq, m):
    scale = 1.0 / math.sqrt(A)

    def body(q_ref, k_ref, v_ref, wo_ref, out_ref):
        s = _dot_nt(q_ref[...], k_ref[...]) * scale
        mx = jnp.max(s, axis=1, keepdims=True)
        p = jnp.exp(s - mx)
        l = jnp.sum(p, axis=1, keepdims=True)
        a = p / l
        o = _dot(a, v_ref[...])
        out_ref[...] = _dot(o, wo_ref[...])

    return pl.pallas_call(
        body,
        grid=(nq // bq,),
        in_specs=[pl.BlockSpec((bq, A), lambda i: (i, 0)),
                  pl.BlockSpec((m, A), lambda i: (0, 0)),
                  pl.BlockSpec((m, A), lambda i: (0, 0)),
                  pl.BlockSpec((A, D), lambda i: (0, 0))],
        out_specs=pl.BlockSpec((bq, D), lambda i: (i, 0)),
        out_shape=jax.ShapeDtypeStruct((nq, D), jnp.float32))


def _make_update_lig(n, bn):
    def body(x_ref, m0_ref, m1_ref, a0_ref, a1_ref, cr_ref, pos_ref, w1_ref,
             bn1_ref, wn2_ref, bn2_ref, xo_ref, po_ref):
        x = x_ref[...]
        msum = m0_ref[...] + m1_ref[...]
        aux = a0_ref[...] + a1_ref[...]
        deg = jnp.maximum(aux[:, 3:4], 1.0)
        agg = msum / deg
        cat = jnp.concatenate([x, agg, cr_ref[...]], axis=1)
        pre = _dot(cat, w1_ref[...]) + bn1_ref[...]
        h = _silu(pre)
        xo_ref[...] = x + _dot(h, wn2_ref[...]) + bn2_ref[...]
        po_ref[...] = pos_ref[...] + aux[:, :3] / deg

    wspec = pl.BlockSpec((3 * D, D), lambda i: (0, 0))
    w2spec = pl.BlockSpec((D, D), lambda i: (0, 0))
    bspec = pl.BlockSpec((1, D), lambda i: (0, 0))
    nspec = pl.BlockSpec((bn, D), lambda i: (i, 0))
    return pl.pallas_call(
        body,
        grid=(n // bn,),
        in_specs=[nspec, nspec, nspec, nspec, nspec, nspec,
                  pl.BlockSpec((bn, 3), lambda i: (i, 0)),
                  wspec, bspec, w2spec, bspec],
        out_specs=[nspec, pl.BlockSpec((bn, 3), lambda i: (i, 0))],
        out_shape=[jax.ShapeDtypeStruct((n, D), jnp.float32),
                   jax.ShapeDtypeStruct((n, 3), jnp.float32)])


def _make_update_rec(n, bn):
    def body(x_ref, m0_ref, m1_ref, d0_ref, d1_ref, cr_ref, w1_ref,
             bn1_ref, wn2_ref, bn2_ref, xo_ref):
        x = x_ref[...]
        msum = m0_ref[...] + m1_ref[...]
        deg = jnp.maximum(d0_ref[...] + d1_ref[...], 1.0)
        agg = msum / deg
        cat = jnp.concatenate([x, agg, cr_ref[...]], axis=1)
        pre = _dot(cat, w1_ref[...]) + bn1_ref[...]
        h = _silu(pre)
        xo_ref[...] = x + _dot(h, wn2_ref[...]) + bn2_ref[...]

    wspec = pl.BlockSpec((3 * D, D), lambda i: (0, 0))
    w2spec = pl.BlockSpec((D, D), lambda i: (0, 0))
    bspec = pl.BlockSpec((1, D), lambda i: (0, 0))
    nspec = pl.BlockSpec((bn, D), lambda i: (i, 0))
    dspec = pl.BlockSpec((bn, 1), lambda i: (i, 0))
    return pl.pallas_call(
        body,
        grid=(n // bn,),
        in_specs=[nspec, nspec, nspec, dspec, dspec, nspec,
                  wspec, bspec, w2spec, bspec],
        out_specs=nspec,
        out_shape=jax.ShapeDtypeStruct((n, D), jnp.float32))


def _make_keypoints(n):
    scale = 1.0 / math.sqrt(D)

    def body(x_ref, pos_ref, q_ref, wk_ref, kp_ref):
        x = x_ref[...]
        xk = _dot(x, wk_ref[...])
        s = _dot_nt(q_ref[...], xk) * scale
        mx = jnp.max(s, axis=1, keepdims=True)
        p = jnp.exp(s - mx)
        a = p / jnp.sum(p, axis=1, keepdims=True)
        kp_ref[...] = _dot(a, pos_ref[...])

    return pl.pallas_call(
        body,
        in_specs=[pl.BlockSpec((n, D), lambda: (0, 0)),
                  pl.BlockSpec((n, 3), lambda: (0, 0)),
                  pl.BlockSpec((KQ, D), lambda: (0, 0)),
                  pl.BlockSpec((D, D), lambda: (0, 0))],
        out_specs=pl.BlockSpec((KQ, 3), lambda: (0, 0)),
        out_shape=jax.ShapeDtypeStruct((KQ, 3), jnp.float32))


def _kabsch_body(pk_ref, qk_ref, lp_ref, n4m_ref, r3m_ref, r_ref, t_ref,
                 pred_ref):
    P = pk_ref[...]
    Q = qk_ref[...]
    pm = jnp.mean(P, axis=0, keepdims=True)
    qm = jnp.mean(Q, axis=0, keepdims=True)
    H = lax.dot_general(P - pm, Q - qm, (((0,), (0,)), ((), ())),
                        preferred_element_type=jnp.float32, precision=_PREC)

    def el(i, j):
        return H[i:i + 1, j:j + 1]

    coeffs = [el(0, 0), el(1, 1), el(2, 2), el(1, 2), el(2, 1),
              el(2, 0), el(0, 2), el(0, 1), el(1, 0)]
    shift = 2.0 * jnp.sum(jnp.abs(H)) + 0.001
    m4 = shift * n4m_ref[9]
    for k, cf in enumerate(coeffs):
        m4 = m4 + cf * n4m_ref[k]

    m4 = m4 / jnp.sqrt(jnp.sum(m4 * m4))

    def it(i, m):
        m = _dot(m, m, _HI)
        return m / jnp.sqrt(jnp.sum(m * m))

    m4 = lax.fori_loop(0, 30, it, m4)
    q = _dot(m4, jnp.full((4, 1), 0.5, jnp.float32), _HI)
    q = q / jnp.sqrt(jnp.sum(q * q))
    w = q[0:1, 0:1]
    x = q[1:2, 0:1]
    y = q[2:3, 0:1]
    z = q[3:4, 0:1]
    one = jnp.ones((1, 1), jnp.float32)
    terms = [one - 2.0 * (y * y + z * z), 2.0 * (x * y - w * z),
             2.0 * (x * z + w * y), 2.0 * (x * y + w * z),
             one - 2.0 * (x * x + z * z), 2.0 * (y * z - w * x),
             2.0 * (x * z - w * y), 2.0 * (y * z + w * x),
             one - 2.0 * (x * x + y * y)]
    R = terms[0] * r3m_ref[0]
    for k in range(1, 9):
        R = R + terms[k] * r3m_ref[k]
    Rb = _b16(R)

    def dot3_nt(a):
        ab = _b16(a)
        return ((ab[:, 0:1] * Rb[:, 0:1].T + ab[:, 1:2] * Rb[:, 1:2].T)
                + ab[:, 2:3] * Rb[:, 2:3].T)

    t_row = qm - dot3_nt(pm)
    r_ref[...] = R
    t_ref[...] = t_row
    pred_ref[...] = dot3_nt(lp_ref[...]) + t_row


_kabsch_call = pl.pallas_call(
    _kabsch_body,
    in_specs=[pl.BlockSpec((KQ, 3), lambda: (0, 0)),
              pl.BlockSpec((KQ, 3), lambda: (0, 0)),
              pl.BlockSpec((NLIG, 3), lambda: (0, 0)),
              pl.BlockSpec((10, 4, 4), lambda: (0, 0, 0)),
              pl.BlockSpec((9, 3, 3), lambda: (0, 0, 0))],
    out_specs=[pl.BlockSpec((3, 3), lambda: (0, 0)),
               pl.BlockSpec((1, 3), lambda: (0, 0)),
               pl.BlockSpec((NLIG, 3), lambda: (0, 0))],
    out_shape=[jax.ShapeDtypeStruct((3, 3), jnp.float32),
               jax.ShapeDtypeStruct((1, 3), jnp.float32),
               jax.ShapeDtypeStruct((NLIG, 3), jnp.float32)])


def _kabsch_masks():
    n4m = [
        [[1, 0, 0, 0], [0, 1, 0, 0], [0, 0, -1, 0], [0, 0, 0, -1]],
        [[1, 0, 0, 0], [0, -1, 0, 0], [0, 0, 1, 0], [0, 0, 0, -1]],
        [[1, 0, 0, 0], [0, -1, 0, 0], [0, 0, -1, 0], [0, 0, 0, 1]],
        [[0, 1, 0, 0], [1, 0, 0, 0], [0, 0, 0, 1], [0, 0, 1, 0]],
        [[0, -1, 0, 0], [-1, 0, 0, 0], [0, 0, 0, 1], [0, 0, 1, 0]],
        [[0, 0, 1, 0], [0, 0, 0, 1], [1, 0, 0, 0], [0, 1, 0, 0]],
        [[0, 0, -1, 0], [0, 0, 0, 1], [-1, 0, 0, 0], [0, 1, 0, 0]],
        [[0, 0, 0, 1], [0, 0, 1, 0], [0, 1, 0, 0], [1, 0, 0, 0]],
        [[0, 0, 0, -1], [0, 0, 1, 0], [0, 1, 0, 0], [-1, 0, 0, 0]],
        [[1, 0, 0, 0], [0, 1, 0, 0], [0, 0, 1, 0], [0, 0, 0, 1]],
    ]
    r3m = [[[1 if 3 * i + j == k else 0 for j in range(3)]
            for i in range(3)] for k in range(9)]
    return jnp.asarray(n4m, jnp.float32), jnp.asarray(r3m, jnp.float32)


class _LazyKernel:
    def __init__(self, factory, *args):
        self._factory = factory
        self._args = args
        self._fn = None

    def __call__(self, *xs):
        if self._fn is None:
            self._fn = self._factory(*self._args)
        return self._fn(*xs)


_gather_l = _LazyKernel(_make_gather, ELP, ELP // NW // CHK)
_gather_r = _LazyKernel(_make_gather, ERP, ERP // NW // CHK)
_scatter_l = _LazyKernel(_make_scatter, NL16, ELP // NW // CHK)
_scatter_r = _LazyKernel(_make_scatter, NR16, ERP // NW // CHK)
_edge_mlp_l = _make_edge_mlp(ELP, 2048, True)
_edge_mlp_r = _make_edge_mlp(ERP, 2048, False)
_qkv_l = _make_qkv(NLIG, 1000)
_qkv_r = _make_qkv(NREC, 1000)
_attn_lr = _make_attn(NLIG, 400, NREC)
_attn_rl = _make_attn(NREC, 1000, NLIG)
_update_l = _make_update_lig(NLIG, 400)
_update_r = _make_update_rec(NREC, 1000)
_keypoints_l = _make_keypoints(NLIG)
_keypoints_r = _make_keypoints(NREC)


def _edge_plumbing(edge_index, n, ep, e):
    src = edge_index[0]
    dst = edge_index[1]
    pad = ep - e
    gidx_d = jnp.concatenate([dst, jnp.zeros((pad,), jnp.int32)])
    gidx_s = jnp.concatenate([src, jnp.zeros((pad,), jnp.int32)])
    sidx = jnp.concatenate([dst, jnp.full((pad,), n, jnp.int32)])
    shape = (NW, ep // NW // CHK, CHK)
    return gidx_d.reshape(shape), gidx_s.reshape(shape), sidx.reshape(shape)


def _node_table(x, pos, n):
    return jnp.concatenate(
        [x, pos, jnp.zeros((n, TW - D - 3), jnp.float32)], axis=1)


def kernel(lig_x, lig_pos, lig_edge_index, rec_x, rec_pos, rec_edge_index,
           params):
    gl_d, gl_s, sl = _edge_plumbing(lig_edge_index, NLIG, ELP, ELIG)
    gr_d, gr_s, sr = _edge_plumbing(rec_edge_index, NREC, ERP, EREC)
    zl = jnp.zeros((NL16, VW), jnp.float32)
    zr = jnp.zeros((NR16, VW), jnp.float32)

    ones_r = jnp.zeros((ERP, VW), jnp.float32).at[:EREC, 0].set(1.0)
    degp_r = _scatter_r(ones_r, sr, zr)[0]
    deg_r0 = degp_r[0, :NREC, 0:1]
    deg_r1 = degp_r[1, :NREC, 0:1]

    lx, lp = lig_x, lig_pos
    rx, rp = rec_x, rec_pos
    for p in params['layers']:
        we1 = p['We1'][:2 * D]
        wdist = p['We1'][2 * D:2 * D + 1]
        be1 = p['be1'][None, :]
        be2 = p['be2'][None, :]
        bc1 = p['bc1'][None, :]
        wc2 = p['Wc2'].T
        bc2 = p['bc2'][None, :]
        wqkv = jnp.concatenate([p['Wq'], p['Wk'], p['Wv']], axis=1)
        bn1 = p['bn1'][None, :]
        bn2 = p['bn2'][None, :]

        tab_l = _node_table(lx, lp, NLIG)
        tab_r = _node_table(rx, rp, NREC)
        gd_l, gs_l = _gather_l(tab_l, gl_d, gl_s)
        gd_r, gs_r = _gather_r(tab_r, gr_d, gr_s)
        m_l, aux_l = _edge_mlp_l(gd_l, gs_l, we1, wdist, be1, p['We2'], be2,
                                 p['Wc1'], bc1, wc2, bc2)
        m_r = _edge_mlp_r(gd_r, gs_r, we1, wdist, be1, p['We2'], be2,
                          p['Wc1'], bc1, wc2, bc2)
        mparts_l = _scatter_l(m_l, sl, zl)[0]
        aparts_l = _scatter_l(aux_l, sl, zl)[0]
        mparts_r = _scatter_r(m_r, sr, zr)[0]

        qkv_l = _qkv_l(lx, wqkv)
        qkv_r = _qkv_r(rx, wqkv)
        cross_l = _attn_lr(qkv_l[:, :A], qkv_r[:, A:2 * A],
                           qkv_r[:, 2 * A:], p['Wo'])
        cross_r = _attn_rl(qkv_r[:, :A], qkv_l[:, A:2 * A],
                           qkv_l[:, 2 * A:], p['Wo'])

        lx, lp = _update_l(lx, mparts_l[0, :NLIG], mparts_l[1, :NLIG],
                           aparts_l[0, :NLIG], aparts_l[1, :NLIG],
                           cross_l, lp, p['Wn1'], bn1, p['Wn2'], bn2)
        rx = _update_r(rx, mparts_r[0, :NREC], mparts_r[1, :NREC],
                       deg_r0, deg_r1, cross_r,
                       p['Wn1'], bn1, p['Wn2'], bn2)

    kp_l = params['kp_lig']
    kp_r = params['kp_rec']
    lig_kp = _keypoints_l(lx, lp, kp_l['Q'], kp_l['Wk'])
    rec_kp = _keypoints_r(rx, rp, kp_r['Q'], kp_r['Wk'])
    n4m, r3m = _kabsch_masks()
    R, t_row, lig_pos_pred = _kabsch_call(lig_kp, rec_kp, lp, n4m, r3m)
    return (lig_pos_pred, lx, rx, lig_kp, rec_kp, R, t_row.reshape(3))
